# baseline (speedup 1.0000x reference)
import functools

import jax
import jax.numpy as jnp
from jax.experimental import pallas as pl
from jax.experimental.pallas import tpu as pltpu

N = 4096
NN = 2 * N
D = 128
EPS2 = 2e-10


def _layer_body(x_ref, af_ref, ab_ref, c1_ref, c2_ref, w1_ref, w2_ref, wr_ref,
                br_ref, o_ref):
    x = x_ref[...]
    c1 = jnp.maximum(c1_ref[...], 1.0)
    c2 = jnp.maximum(c2_ref[...], 1.0)
    af = af_ref[...] / c1
    ab = ab_ref[...] / c2
    acc = jax.lax.dot_general(x, wr_ref[...], (((1,), (0,)), ((), ())),
                              preferred_element_type=jnp.float32)
    acc += jax.lax.dot_general(af, w1_ref[...], (((1,), (0,)), ((), ())),
                               preferred_element_type=jnp.float32)
    acc += jax.lax.dot_general(ab, w2_ref[...], (((1,), (0,)), ((), ())),
                               preferred_element_type=jnp.float32)
    o_ref[...] = jnp.maximum(acc + br_ref[...], 0.0)


def _layer(x, af, ab, c1, c2, w1, w2, wr, br, rows=512):
    t = NN // rows
    full = lambda i: (0, 0)
    byrow = lambda i: (i, 0)
    return pl.pallas_call(
        _layer_body,
        grid=(t,),
        in_specs=[
            pl.BlockSpec((rows, D), byrow),
            pl.BlockSpec((rows, D), byrow),
            pl.BlockSpec((rows, D), byrow),
            pl.BlockSpec((rows, 1), byrow),
            pl.BlockSpec((rows, 1), byrow),
            pl.BlockSpec((D, D), full),
            pl.BlockSpec((D, D), full),
            pl.BlockSpec((D, D), full),
            pl.BlockSpec((1, D), full),
        ],
        out_specs=pl.BlockSpec((rows, D), byrow),
        out_shape=jax.ShapeDtypeStruct((NN, D), jnp.float32),
    )(x, af, ab, c1, c2, w1, w2, wr, br)


def _final_body(x_ref, h1_ref, h2_ref, f0_ref, f1_ref, f2_ref, fb_ref, o_ref):
    acc = jax.lax.dot_general(x_ref[...], f0_ref[...], (((1,), (0,)), ((), ())),
                              preferred_element_type=jnp.float32)
    acc += jax.lax.dot_general(h1_ref[...], f1_ref[...], (((1,), (0,)), ((), ())),
                               preferred_element_type=jnp.float32)
    acc += jax.lax.dot_general(h2_ref[...], f2_ref[...], (((1,), (0,)), ((), ())),
                               preferred_element_type=jnp.float32)
    o_ref[...] = acc + fb_ref[...]


def _final(x, h1, h2, f0, f1, f2, fb, rows=512):
    t = NN // rows
    full = lambda i: (0, 0)
    byrow = lambda i: (i, 0)
    return pl.pallas_call(
        _final_body,
        grid=(t,),
        in_specs=[
            pl.BlockSpec((rows, D), byrow),
            pl.BlockSpec((rows, D), byrow),
            pl.BlockSpec((rows, D), byrow),
            pl.BlockSpec((D, D), full),
            pl.BlockSpec((D, D), full),
            pl.BlockSpec((D, D), full),
            pl.BlockSpec((1, D), full),
        ],
        out_specs=pl.BlockSpec((rows, D), byrow),
        out_shape=jax.ShapeDtypeStruct((NN, D), jnp.float32),
    )(x, h1, h2, f0, f1, f2, fb)


def _sinkhorn_body(hs_ref, ht_ref, o_ref, u_ref, v_ref, acc_ref, *, rows):
    p = pl.program_id(0)
    t = pl.program_id(1)
    nt = pl.num_programs(1)
    m = jnp.exp(
        2.0 * jax.lax.dot_general(hs_ref[...], ht_ref[...],
                                  (((1,), (1,)), ((), ())),
                                  preferred_element_type=jnp.float32) + EPS2)

    is_colpass = (p == 0) | (p == 2) | (p == 4)
    is_rowpass = (p == 1) | (p == 3)

    @pl.when(is_colpass)
    def _():
        @pl.when(t == 0)
        def _():
            acc_ref[...] = jnp.zeros_like(acc_ref)

        w = jnp.where(p == 0, jnp.ones((rows, 1), jnp.float32),
                      u_ref[pl.ds(t * rows, rows), :])
        acc_ref[...] += jnp.sum(m * w, axis=0, keepdims=True)

        @pl.when(t == nt - 1)
        def _():
            v_ref[...] = 1.0 / acc_ref[...]

    @pl.when(is_rowpass)
    def _():
        s = jnp.sum(m * v_ref[...], axis=1, keepdims=True)
        u_ref[pl.ds(t * rows, rows), :] = 1.0 / s

    @pl.when(p == 5)
    def _():
        o_ref[...] = m * u_ref[pl.ds(t * rows, rows), :] * v_ref[...]


def _sinkhorn(hs, ht, rows=256):
    t = N // rows
    kern = functools.partial(_sinkhorn_body, rows=rows)
    return pl.pallas_call(
        kern,
        grid=(6, t),
        in_specs=[
            pl.BlockSpec((rows, D), lambda p, i: (i, 0)),
            pl.BlockSpec((N, D), lambda p, i: (0, 0)),
        ],
        out_specs=pl.BlockSpec((rows, N),
                               lambda p, i: (jnp.where(p == 5, i, 0), 0)),
        out_shape=jax.ShapeDtypeStruct((N, N), jnp.float32),
        scratch_shapes=[
            pltpu.VMEM((N, 1), jnp.float32),
            pltpu.VMEM((1, N), jnp.float32),
            pltpu.VMEM((1, N), jnp.float32),
        ],
    )(hs, ht)


def _aggregate(x, src, dst):
    ones = jnp.ones((src.shape[0],), jnp.float32)
    af = jax.ops.segment_sum(x[src], dst, num_segments=NN)
    ab = jax.ops.segment_sum(x[dst], src, num_segments=NN)
    c1 = jax.ops.segment_sum(ones, dst, num_segments=NN)[:, None]
    c2 = jax.ops.segment_sum(ones, src, num_segments=NN)[:, None]
    return af, ab, c1, c2


def kernel(x_s, x_t, edges, edget, Hs, Gs, Ht, Gt, dedges, dws, dedget, dwt,
           l1W0, l2W0, rW0, rb0, l1W1, l2W1, rW1, rb1, finW, finb):
    x = jnp.concatenate([x_s, x_t], axis=0)
    src = jnp.concatenate([edges[0], edget[0] + N])
    dst = jnp.concatenate([edges[1], edget[1] + N])

    af, ab, c1, c2 = _aggregate(x, src, dst)
    h1 = _layer(x, af, ab, c1, c2, l1W0, l2W0, rW0, rb0[None, :])

    af, ab, _, _ = _aggregate(h1, src, dst)
    h2 = _layer(h1, af, ab, c1, c2, l1W1, l2W1, rW1, rb1[None, :])

    f0, f1, f2 = finW[:D], finW[D:2 * D], finW[2 * D:]
    h = _final(x, h1, h2, f0, f1, f2, finb[None, :])

    return _sinkhorn(h[:N], h[N:])


# speedup vs baseline: 4.1016x; 4.1016x over previous
import functools

import jax
from jax import lax
import jax.numpy as jnp
from jax.experimental import pallas as pl
from jax.experimental.pallas import tpu as pltpu
from jax.experimental.pallas import tpu_sc as plsc

N = 4096
NN = 2 * N
D = 128
E = 65536
EPS2 = 2e-10

NTILES = 16
EPT = E // NTILES
CH = 128
NCHUNK = EPT // CH
ROWS_PT = N // NTILES


def _layer_body(x_ref, af_ref, ab_ref, c1_ref, c2_ref, w1_ref, w2_ref, wr_ref,
                br_ref, o_ref):
    x = x_ref[...]
    c1 = jnp.maximum(c1_ref[...][:, :1], 1.0)
    c2 = jnp.maximum(c2_ref[...][:, :1], 1.0)
    af = af_ref[...] / c1
    ab = ab_ref[...] / c2
    acc = jax.lax.dot_general(x, wr_ref[...], (((1,), (0,)), ((), ())),
                              preferred_element_type=jnp.float32)
    acc += jax.lax.dot_general(af, w1_ref[...], (((1,), (0,)), ((), ())),
                               preferred_element_type=jnp.float32)
    acc += jax.lax.dot_general(ab, w2_ref[...], (((1,), (0,)), ((), ())),
                               preferred_element_type=jnp.float32)
    o_ref[...] = jnp.maximum(acc + br_ref[...], 0.0)


def _layer(x, af, ab, c1, c2, w1, w2, wr, br, rows=512):
    t = NN // rows
    full = lambda i: (0, 0)
    byrow = lambda i: (i, 0)
    return pl.pallas_call(
        _layer_body,
        grid=(t,),
        in_specs=[
            pl.BlockSpec((rows, D), byrow),
            pl.BlockSpec((rows, D), byrow),
            pl.BlockSpec((rows, D), byrow),
            pl.BlockSpec((rows, D), byrow),
            pl.BlockSpec((rows, D), byrow),
            pl.BlockSpec((D, D), full),
            pl.BlockSpec((D, D), full),
            pl.BlockSpec((D, D), full),
            pl.BlockSpec((1, D), full),
        ],
        out_specs=pl.BlockSpec((rows, D), byrow),
        out_shape=jax.ShapeDtypeStruct((NN, D), jnp.float32),
    )(x, af, ab, c1, c2, w1, w2, wr, br)


def _final_body(x_ref, h1_ref, h2_ref, f0_ref, f1_ref, f2_ref, fb_ref, o_ref):
    acc = jax.lax.dot_general(x_ref[...], f0_ref[...], (((1,), (0,)), ((), ())),
                              preferred_element_type=jnp.float32)
    acc += jax.lax.dot_general(h1_ref[...], f1_ref[...], (((1,), (0,)), ((), ())),
                               preferred_element_type=jnp.float32)
    acc += jax.lax.dot_general(h2_ref[...], f2_ref[...], (((1,), (0,)), ((), ())),
                               preferred_element_type=jnp.float32)
    o_ref[...] = acc + fb_ref[...]


def _final(x, h1, h2, f0, f1, f2, fb, rows=512):
    t = NN // rows
    full = lambda i: (0, 0)
    byrow = lambda i: (i, 0)
    return pl.pallas_call(
        _final_body,
        grid=(t,),
        in_specs=[
            pl.BlockSpec((rows, D), byrow),
            pl.BlockSpec((rows, D), byrow),
            pl.BlockSpec((rows, D), byrow),
            pl.BlockSpec((D, D), full),
            pl.BlockSpec((D, D), full),
            pl.BlockSpec((D, D), full),
            pl.BlockSpec((1, D), full),
        ],
        out_specs=pl.BlockSpec((rows, D), byrow),
        out_shape=jax.ShapeDtypeStruct((NN, D), jnp.float32),
    )(x, h1, h2, f0, f1, f2, fb)


def _sinkhorn_body(hs_ref, ht_ref, o_ref, u_ref, v_ref, acc_ref, *, rows):
    p = pl.program_id(0)
    t = pl.program_id(1)
    nt = pl.num_programs(1)
    m = jnp.exp(
        2.0 * jax.lax.dot_general(hs_ref[...], ht_ref[...],
                                  (((1,), (1,)), ((), ())),
                                  preferred_element_type=jnp.float32) + EPS2)

    is_colpass = (p == 0) | (p == 2) | (p == 4)
    is_rowpass = (p == 1) | (p == 3)

    @pl.when(is_colpass)
    def _():
        @pl.when(t == 0)
        def _():
            acc_ref[...] = jnp.zeros_like(acc_ref)

        w = jnp.where(p == 0, jnp.ones((rows, 1), jnp.float32),
                      u_ref[pl.ds(t * rows, rows), :])
        acc_ref[...] += jnp.sum(m * w, axis=0, keepdims=True)

        @pl.when(t == nt - 1)
        def _():
            v_ref[...] = 1.0 / acc_ref[...]

    @pl.when(is_rowpass)
    def _():
        s = jnp.sum(m * v_ref[...], axis=1, keepdims=True)
        u_ref[pl.ds(t * rows, rows), :] = 1.0 / s

    @pl.when(p == 5)
    def _():
        o_ref[...] = m * u_ref[pl.ds(t * rows, rows), :] * v_ref[...]


def _sinkhorn(hs, ht, rows=256):
    t = N // rows
    kern = functools.partial(_sinkhorn_body, rows=rows)
    return pl.pallas_call(
        kern,
        grid=(6, t),
        in_specs=[
            pl.BlockSpec((rows, D), lambda p, i: (i, 0)),
            pl.BlockSpec((N, D), lambda p, i: (0, 0)),
        ],
        out_specs=pl.BlockSpec((rows, N),
                               lambda p, i: (jnp.where(p == 5, i, 0), 0)),
        out_shape=jax.ShapeDtypeStruct((N, N), jnp.float32),
        scratch_shapes=[
            pltpu.VMEM((N, 1), jnp.float32),
            pltpu.VMEM((1, N), jnp.float32),
            pltpu.VMEM((1, N), jnp.float32),
        ],
    )(hs, ht)


_SC_MESH = plsc.VectorSubcoreMesh(core_axis_name="c", subcore_axis_name="s")
_F32 = jnp.float32


def _agg_kernel_body(x_hbm, sl_hbm, dl_hbm, sg_hbm, dg_hbm, z_hbm,
                     af_hbm, ab_hbm,
                     sv, dv, sgv, dgv, rows1, rows2, accf, accb, sem):
    c = lax.axis_index("c")
    w = lax.axis_index("s")
    sl = pl.ds(w * ROWS_PT, ROWS_PT)

    pltpu.sync_copy(z_hbm, accf.at[sl])
    pltpu.sync_copy(z_hbm, accb.at[sl])
    plsc.subcore_barrier()

    ebase = c * E + w * EPT

    @pl.loop(0, NCHUNK)
    def _(k):
        base = ebase + k * CH
        pltpu.sync_copy(sl_hbm.at[pl.ds(base, CH)], sv)
        pltpu.sync_copy(dl_hbm.at[pl.ds(base, CH)], dv)
        pltpu.sync_copy(sg_hbm.at[pl.ds(base, CH)], sgv)
        pltpu.sync_copy(dg_hbm.at[pl.ds(base, CH)], dgv)

        pltpu.async_copy(x_hbm.at[sgv], rows1, sem).wait()
        pltpu.sync_copy(rows1, accf.at[dv], add=True)
        pltpu.async_copy(x_hbm.at[dgv], rows2, sem).wait()
        pltpu.sync_copy(rows2, accb.at[sv], add=True)

    plsc.subcore_barrier()

    orow = pl.ds(c * N + w * ROWS_PT, ROWS_PT)
    pltpu.sync_copy(accf.at[sl], af_hbm.at[orow])
    pltpu.sync_copy(accb.at[sl], ab_hbm.at[orow])


_agg = pl.kernel(
    _agg_kernel_body,
    out_type=[jax.ShapeDtypeStruct((NN, D), _F32),
              jax.ShapeDtypeStruct((NN, D), _F32)],
    mesh=_SC_MESH,
    scratch_types=[
        pltpu.VMEM((CH,), jnp.int32),
        pltpu.VMEM((CH,), jnp.int32),
        pltpu.VMEM((CH,), jnp.int32),
        pltpu.VMEM((CH,), jnp.int32),
        pltpu.VMEM((CH, D), _F32),
        pltpu.VMEM((CH, D), _F32),
        pltpu.VMEM_SHARED((N, D), _F32),
        pltpu.VMEM_SHARED((N, D), _F32),
        pltpu.SemaphoreType.DMA,
    ],
)


def _counts_kernel_body(sl_hbm, dl_hbm, z_hbm, o_hbm, cf_hbm, cb_hbm,
                        sv, dv, ones_v, cntf, cntb):
    c = lax.axis_index("c")
    w = lax.axis_index("s")
    sl = pl.ds(w * ROWS_PT, ROWS_PT)

    pltpu.sync_copy(z_hbm, cntf.at[sl])
    pltpu.sync_copy(z_hbm, cntb.at[sl])
    pltpu.sync_copy(o_hbm, ones_v)
    plsc.subcore_barrier()

    ebase = c * E + w * EPT

    @pl.loop(0, NCHUNK)
    def _(k):
        base = ebase + k * CH
        pltpu.sync_copy(sl_hbm.at[pl.ds(base, CH)], sv)
        pltpu.sync_copy(dl_hbm.at[pl.ds(base, CH)], dv)
        pltpu.sync_copy(ones_v, cntf.at[dv], add=True)
        pltpu.sync_copy(ones_v, cntb.at[sv], add=True)

    plsc.subcore_barrier()

    orow = pl.ds(c * N + w * ROWS_PT, ROWS_PT)
    pltpu.sync_copy(cntf.at[sl], cf_hbm.at[orow])
    pltpu.sync_copy(cntb.at[sl], cb_hbm.at[orow])


_counts = pl.kernel(
    _counts_kernel_body,
    out_type=[jax.ShapeDtypeStruct((NN, D), _F32),
              jax.ShapeDtypeStruct((NN, D), _F32)],
    mesh=_SC_MESH,
    scratch_types=[
        pltpu.VMEM((CH,), jnp.int32),
        pltpu.VMEM((CH,), jnp.int32),
        pltpu.VMEM((CH, D), _F32),
        pltpu.VMEM_SHARED((N, D), _F32),
        pltpu.VMEM_SHARED((N, D), _F32),
    ],
)


def kernel(x_s, x_t, edges, edget, Hs, Gs, Ht, Gt, dedges, dws, dedget, dwt,
           l1W0, l2W0, rW0, rb0, l1W1, l2W1, rW1, rb1, finW, finb):
    x = jnp.concatenate([x_s, x_t], axis=0)
    src_l = jnp.concatenate([edges[0], edget[0]]).astype(jnp.int32)
    dst_l = jnp.concatenate([edges[1], edget[1]]).astype(jnp.int32)
    src_g = jnp.concatenate([edges[0], edget[0] + N]).astype(jnp.int32)
    dst_g = jnp.concatenate([edges[1], edget[1] + N]).astype(jnp.int32)
    z = jnp.zeros((ROWS_PT, D), jnp.float32)
    on = jnp.ones((CH, D), jnp.float32)

    cf, cb = _counts(src_l, dst_l, z, on)
    af, ab = _agg(x, src_l, dst_l, src_g, dst_g, z)
    h1 = _layer(x, af, ab, cf, cb, l1W0, l2W0, rW0, rb0[None, :])

    af, ab = _agg(h1, src_l, dst_l, src_g, dst_g, z)
    h2 = _layer(h1, af, ab, cf, cb, l1W1, l2W1, rW1, rb1[None, :])

    f0, f1, f2 = finW[:D], finW[D:2 * D], finW[2 * D:]
    h = _final(x, h1, h2, f0, f1, f2, finb[None, :])

    return _sinkhorn(h[:N], h[N:])


# speedup vs baseline: 4.6262x; 1.1279x over previous
import functools

import jax
from jax import lax
import jax.numpy as jnp
from jax.experimental import pallas as pl
from jax.experimental.pallas import tpu as pltpu
from jax.experimental.pallas import tpu_sc as plsc

N = 4096
NN = 2 * N
D = 128
E = 65536
EPS2 = 2e-10

NTILES = 16
EPT = E // NTILES
CH = 128
NCHUNK = EPT // CH
CHA = 64
NCHUNKA = EPT // CHA
ROWS_PT = N // NTILES


def _layer_body(x_ref, af_ref, ab_ref, c1_ref, c2_ref, w1_ref, w2_ref, wr_ref,
                br_ref, o_ref):
    x = x_ref[...]
    c1 = jnp.maximum(c1_ref[...][:, :1], 1.0)
    c2 = jnp.maximum(c2_ref[...][:, :1], 1.0)
    af = af_ref[...] / c1
    ab = ab_ref[...] / c2
    acc = jax.lax.dot_general(x, wr_ref[...], (((1,), (0,)), ((), ())),
                              preferred_element_type=jnp.float32)
    acc += jax.lax.dot_general(af, w1_ref[...], (((1,), (0,)), ((), ())),
                               preferred_element_type=jnp.float32)
    acc += jax.lax.dot_general(ab, w2_ref[...], (((1,), (0,)), ((), ())),
                               preferred_element_type=jnp.float32)
    o_ref[...] = jnp.maximum(acc + br_ref[...], 0.0)


def _layer(x, af, ab, c1, c2, w1, w2, wr, br, rows=512):
    t = NN // rows
    full = lambda i: (0, 0)
    byrow = lambda i: (i, 0)
    return pl.pallas_call(
        _layer_body,
        grid=(t,),
        in_specs=[
            pl.BlockSpec((rows, D), byrow),
            pl.BlockSpec((rows, D), byrow),
            pl.BlockSpec((rows, D), byrow),
            pl.BlockSpec((rows, D), byrow),
            pl.BlockSpec((rows, D), byrow),
            pl.BlockSpec((D, D), full),
            pl.BlockSpec((D, D), full),
            pl.BlockSpec((D, D), full),
            pl.BlockSpec((1, D), full),
        ],
        out_specs=pl.BlockSpec((rows, D), byrow),
        out_shape=jax.ShapeDtypeStruct((NN, D), jnp.float32),
    )(x, af, ab, c1, c2, w1, w2, wr, br)


def _final_body(x_ref, h1_ref, h2_ref, f0_ref, f1_ref, f2_ref, fb_ref, o_ref):
    acc = jax.lax.dot_general(x_ref[...], f0_ref[...], (((1,), (0,)), ((), ())),
                              preferred_element_type=jnp.float32)
    acc += jax.lax.dot_general(h1_ref[...], f1_ref[...], (((1,), (0,)), ((), ())),
                               preferred_element_type=jnp.float32)
    acc += jax.lax.dot_general(h2_ref[...], f2_ref[...], (((1,), (0,)), ((), ())),
                               preferred_element_type=jnp.float32)
    o_ref[...] = (acc + fb_ref[...]).astype(jnp.bfloat16)


def _final(x, h1, h2, f0, f1, f2, fb, rows=512):
    t = NN // rows
    full = lambda i: (0, 0)
    byrow = lambda i: (i, 0)
    return pl.pallas_call(
        _final_body,
        grid=(t,),
        in_specs=[
            pl.BlockSpec((rows, D), byrow),
            pl.BlockSpec((rows, D), byrow),
            pl.BlockSpec((rows, D), byrow),
            pl.BlockSpec((D, D), full),
            pl.BlockSpec((D, D), full),
            pl.BlockSpec((D, D), full),
            pl.BlockSpec((1, D), full),
        ],
        out_specs=pl.BlockSpec((rows, D), byrow),
        out_shape=jax.ShapeDtypeStruct((NN, D), jnp.bfloat16),
    )(x, h1, h2, f0, f1, f2, fb)


def _sinkhorn_body(hs_ref, ht_ref, o_ref, u_ref, v_ref, acc_ref, *, rows):
    p = pl.program_id(0)
    t = pl.program_id(1)
    nt = pl.num_programs(1)
    m = jnp.exp(
        2.0 * jax.lax.dot_general(hs_ref[...], ht_ref[...],
                                  (((1,), (1,)), ((), ())),
                                  preferred_element_type=jnp.float32) + EPS2)

    is_colpass = (p == 0) | (p == 2) | (p == 4)
    is_rowpass = (p == 1) | (p == 3)

    @pl.when(is_colpass)
    def _():
        @pl.when(t == 0)
        def _():
            acc_ref[...] = jnp.zeros_like(acc_ref)

        w = jnp.where(p == 0, jnp.ones((rows, 1), jnp.float32),
                      u_ref[pl.ds(t * rows, rows), :])
        acc_ref[...] += jnp.sum(m * w, axis=0, keepdims=True)

        @pl.when(t == nt - 1)
        def _():
            v_ref[...] = 1.0 / acc_ref[...]

    @pl.when(is_rowpass)
    def _():
        s = jnp.sum(m * v_ref[...], axis=1, keepdims=True)
        u_ref[pl.ds(t * rows, rows), :] = 1.0 / s

    @pl.when(p == 5)
    def _():
        o_ref[...] = m * u_ref[pl.ds(t * rows, rows), :] * v_ref[...]


def _sinkhorn(hs, ht, rows=512):
    t = N // rows
    kern = functools.partial(_sinkhorn_body, rows=rows)
    return pl.pallas_call(
        kern,
        grid=(6, t),
        in_specs=[
            pl.BlockSpec((rows, D), lambda p, i: (i, 0)),
            pl.BlockSpec((N, D), lambda p, i: (0, 0)),
        ],
        out_specs=pl.BlockSpec((rows, N),
                               lambda p, i: (jnp.where(p == 5, i, 0), 0)),
        out_shape=jax.ShapeDtypeStruct((N, N), jnp.float32),
        scratch_shapes=[
            pltpu.VMEM((N, 1), jnp.float32),
            pltpu.VMEM((1, N), jnp.float32),
            pltpu.VMEM((1, N), jnp.float32),
        ],
    )(hs, ht)


_SC_MESH = plsc.VectorSubcoreMesh(core_axis_name="c", subcore_axis_name="s")
_F32 = jnp.float32


def _agg_kernel_body(x_hbm, sl_hbm, dl_hbm, sg_hbm, dg_hbm, z_hbm,
                     af_hbm, ab_hbm,
                     sva, dva, sgva, dgva, svb, dvb, sgvb, dgvb,
                     r1a, r2a, r1b, r2b, accf, accb,
                     s1a, s2a, s1b, s2b):
    c = lax.axis_index("c")
    w = lax.axis_index("s")
    sl = pl.ds(w * ROWS_PT, ROWS_PT)

    pltpu.sync_copy(z_hbm, accf.at[sl])
    pltpu.sync_copy(z_hbm, accb.at[sl])
    plsc.subcore_barrier()

    ebase = c * E + w * EPT

    def load_idx(k, svx, dvx, sgvx, dgvx):
        base = ebase + k * CHA
        pltpu.sync_copy(sl_hbm.at[pl.ds(base, CHA)], svx)
        pltpu.sync_copy(dl_hbm.at[pl.ds(base, CHA)], dvx)
        pltpu.sync_copy(sg_hbm.at[pl.ds(base, CHA)], sgvx)
        pltpu.sync_copy(dg_hbm.at[pl.ds(base, CHA)], dgvx)

    def start(sgvx, dgvx, r1x, r2x, s1x, s2x):
        pltpu.async_copy(x_hbm.at[sgvx], r1x, s1x)
        pltpu.async_copy(x_hbm.at[dgvx], r2x, s2x)

    def finish(svx, dvx, sgvx, dgvx, r1x, r2x, s1x, s2x):
        pltpu.make_async_copy(x_hbm.at[sgvx], r1x, s1x).wait()
        pltpu.sync_copy(r1x, accf.at[dvx], add=True)
        pltpu.make_async_copy(x_hbm.at[dgvx], r2x, s2x).wait()
        pltpu.sync_copy(r2x, accb.at[svx], add=True)

    A = (sva, dva, sgva, dgva)
    B = (svb, dvb, sgvb, dgvb)
    RA = (r1a, r2a, s1a, s2a)
    RB = (r1b, r2b, s1b, s2b)

    load_idx(0, *A)
    start(sgva, dgva, *RA)

    @pl.loop(0, NCHUNKA - 2, step=2)
    def _(k):
        load_idx(k + 1, *B)
        start(sgvb, dgvb, *RB)
        finish(*A, *RA)
        load_idx(k + 2, *A)
        start(sgva, dgva, *RA)
        finish(*B, *RB)

    load_idx(NCHUNKA - 1, *B)
    start(sgvb, dgvb, *RB)
    finish(*A, *RA)
    finish(*B, *RB)

    plsc.subcore_barrier()

    orow = pl.ds(c * N + w * ROWS_PT, ROWS_PT)
    pltpu.sync_copy(accf.at[sl], af_hbm.at[orow])
    pltpu.sync_copy(accb.at[sl], ab_hbm.at[orow])


_agg = pl.kernel(
    _agg_kernel_body,
    out_type=[jax.ShapeDtypeStruct((NN, D), _F32),
              jax.ShapeDtypeStruct((NN, D), _F32)],
    mesh=_SC_MESH,
    scratch_types=(
        [pltpu.VMEM((CHA,), jnp.int32)] * 8 +
        [pltpu.VMEM((CHA, D), _F32)] * 4 +
        [pltpu.VMEM_SHARED((N, D), _F32),
         pltpu.VMEM_SHARED((N, D), _F32)] +
        [pltpu.SemaphoreType.DMA] * 4
    ),
)


def _counts_kernel_body(sl_hbm, dl_hbm, z_hbm, o_hbm, dep_hbm, cf_hbm, cb_hbm,
                        sv, dv, ones_v, cntf, cntb):
    del dep_hbm
    c = lax.axis_index("c")
    w = lax.axis_index("s")
    sl = pl.ds(w * ROWS_PT, ROWS_PT)

    pltpu.sync_copy(z_hbm, cntf.at[sl])
    pltpu.sync_copy(z_hbm, cntb.at[sl])
    pltpu.sync_copy(o_hbm, ones_v)
    plsc.subcore_barrier()

    ebase = c * E + w * EPT

    @pl.loop(0, NCHUNK)
    def _(k):
        base = ebase + k * CH
        pltpu.sync_copy(sl_hbm.at[pl.ds(base, CH)], sv)
        pltpu.sync_copy(dl_hbm.at[pl.ds(base, CH)], dv)
        pltpu.sync_copy(ones_v, cntf.at[dv], add=True)
        pltpu.sync_copy(ones_v, cntb.at[sv], add=True)

    plsc.subcore_barrier()

    orow = pl.ds(c * N + w * ROWS_PT, ROWS_PT)
    pltpu.sync_copy(cntf.at[sl], cf_hbm.at[orow])
    pltpu.sync_copy(cntb.at[sl], cb_hbm.at[orow])


_counts = pl.kernel(
    _counts_kernel_body,
    out_type=[jax.ShapeDtypeStruct((NN, D), _F32),
              jax.ShapeDtypeStruct((NN, D), _F32)],
    mesh=_SC_MESH,
    scratch_types=[
        pltpu.VMEM((CH,), jnp.int32),
        pltpu.VMEM((CH,), jnp.int32),
        pltpu.VMEM((CH, D), _F32),
        pltpu.VMEM_SHARED((N, D), _F32),
        pltpu.VMEM_SHARED((N, D), _F32),
    ],
)


def kernel(x_s, x_t, edges, edget, Hs, Gs, Ht, Gt, dedges, dws, dedget, dwt,
           l1W0, l2W0, rW0, rb0, l1W1, l2W1, rW1, rb1, finW, finb):
    x = jnp.concatenate([x_s, x_t], axis=0)
    src_l = jnp.concatenate([edges[0], edget[0]]).astype(jnp.int32)
    dst_l = jnp.concatenate([edges[1], edget[1]]).astype(jnp.int32)
    src_g = jnp.concatenate([edges[0], edget[0] + N]).astype(jnp.int32)
    dst_g = jnp.concatenate([edges[1], edget[1] + N]).astype(jnp.int32)
    z = jnp.zeros((ROWS_PT, D), jnp.float32)
    on = jnp.ones((CH, D), jnp.float32)

    af, ab = _agg(x, src_l, dst_l, src_g, dst_g, z)
    cf, cb = _counts(src_l, dst_l, z, on, af)
    h1 = _layer(x, af, ab, cf, cb, l1W0, l2W0, rW0, rb0[None, :])

    af, ab = _agg(h1, src_l, dst_l, src_g, dst_g, z)
    h2 = _layer(h1, af, ab, cf, cb, l1W1, l2W1, rW1, rb1[None, :])

    f0, f1, f2 = finW[:D], finW[D:2 * D], finW[2 * D:]
    h = _final(x, h1, h2, f0, f1, f2, finb[None, :])

    return _sinkhorn(h[:N], h[N:])


# speedup vs baseline: 5.0079x; 1.0825x over previous
import functools

import jax
from jax import lax
import jax.numpy as jnp
from jax.experimental import pallas as pl
from jax.experimental.pallas import tpu as pltpu
from jax.experimental.pallas import tpu_sc as plsc

N = 4096
NN = 2 * N
D = 128
E = 65536
EPS2 = 2e-10

NTILES = 16
EPT = E // NTILES
CH = 128
NCHUNK = EPT // CH
CHA = 64
NCHUNKA = EPT // CHA
ROWS_PT = N // NTILES


def _layer_body(x_ref, af_ref, ab_ref, c1_ref, c2_ref, w1_ref, w2_ref, wr_ref,
                br_ref, o_ref):
    x = x_ref[...]
    c1 = jnp.maximum(c1_ref[...][:, :1], 1.0)
    c2 = jnp.maximum(c2_ref[...][:, :1], 1.0)
    af = af_ref[...] / c1
    ab = ab_ref[...] / c2
    acc = jax.lax.dot_general(x, wr_ref[...], (((1,), (0,)), ((), ())),
                              preferred_element_type=jnp.float32)
    acc += jax.lax.dot_general(af, w1_ref[...], (((1,), (0,)), ((), ())),
                               preferred_element_type=jnp.float32)
    acc += jax.lax.dot_general(ab, w2_ref[...], (((1,), (0,)), ((), ())),
                               preferred_element_type=jnp.float32)
    o_ref[...] = jnp.maximum(acc + br_ref[...], 0.0)


def _layer(x, af, ab, c1, c2, w1, w2, wr, br, rows=512):
    t = NN // rows
    full = lambda i: (0, 0)
    byrow = lambda i: (i, 0)
    return pl.pallas_call(
        _layer_body,
        grid=(t,),
        in_specs=[
            pl.BlockSpec((rows, D), byrow),
            pl.BlockSpec((rows, D), byrow),
            pl.BlockSpec((rows, D), byrow),
            pl.BlockSpec((rows, D), byrow),
            pl.BlockSpec((rows, D), byrow),
            pl.BlockSpec((D, D), full),
            pl.BlockSpec((D, D), full),
            pl.BlockSpec((D, D), full),
            pl.BlockSpec((1, D), full),
        ],
        out_specs=pl.BlockSpec((rows, D), byrow),
        out_shape=jax.ShapeDtypeStruct((NN, D), jnp.float32),
    )(x, af, ab, c1, c2, w1, w2, wr, br)


def _final_body(x_ref, h1_ref, h2_ref, f0_ref, f1_ref, f2_ref, fb_ref, o_ref):
    acc = jax.lax.dot_general(x_ref[...], f0_ref[...], (((1,), (0,)), ((), ())),
                              preferred_element_type=jnp.float32)
    acc += jax.lax.dot_general(h1_ref[...], f1_ref[...], (((1,), (0,)), ((), ())),
                               preferred_element_type=jnp.float32)
    acc += jax.lax.dot_general(h2_ref[...], f2_ref[...], (((1,), (0,)), ((), ())),
                               preferred_element_type=jnp.float32)
    o_ref[...] = ((acc + fb_ref[...]) * 1.6986724).astype(jnp.bfloat16)


def _final(x, h1, h2, f0, f1, f2, fb, rows=512):
    t = NN // rows
    full = lambda i: (0, 0)
    byrow = lambda i: (i, 0)
    return pl.pallas_call(
        _final_body,
        grid=(t,),
        in_specs=[
            pl.BlockSpec((rows, D), byrow),
            pl.BlockSpec((rows, D), byrow),
            pl.BlockSpec((rows, D), byrow),
            pl.BlockSpec((D, D), full),
            pl.BlockSpec((D, D), full),
            pl.BlockSpec((D, D), full),
            pl.BlockSpec((1, D), full),
        ],
        out_specs=pl.BlockSpec((rows, D), byrow),
        out_shape=jax.ShapeDtypeStruct((NN, D), jnp.bfloat16),
    )(x, h1, h2, f0, f1, f2, fb)


def _dotm(a_ref, b_ref):
    return jnp.exp2(jax.lax.dot_general(a_ref[...], b_ref[...],
                                        (((1,), (1,)), ((), ())),
                                        preferred_element_type=jnp.float32))


def _colsum0_body(gs_ref, gt_ref, v0_ref, acc_ref):
    t = pl.program_id(0)
    nt = pl.num_programs(0)
    m = _dotm(gs_ref, gt_ref)

    @pl.when(t == 0)
    def _():
        acc_ref[...] = jnp.zeros_like(acc_ref)

    acc_ref[...] += jnp.sum(m, axis=0, keepdims=True)

    @pl.when(t == nt - 1)
    def _():
        v0_ref[...] = 1.0 / acc_ref[...]


def _uv_body(gs_ref, gt_ref, v0_ref, u_ref, v_ref, vv_ref, acc_ref, *, rows):
    p = pl.program_id(0)
    t = pl.program_id(1)
    nt = pl.num_programs(1)

    @pl.when((p == 0) & (t == 0))
    def _():
        vv_ref[...] = v0_ref[...]

    @pl.when(t == 0)
    def _():
        acc_ref[...] = jnp.zeros_like(acc_ref)

    m = _dotm(gs_ref, gt_ref)
    u_t = 1.0 / jnp.sum(m * vv_ref[...], axis=1, keepdims=True)
    acc_ref[...] += jnp.sum(m * u_t, axis=0, keepdims=True)
    u_ref[...] = u_t

    @pl.when(t == nt - 1)
    def _():
        newv = 1.0 / acc_ref[...]
        vv_ref[...] = newv
        v_ref[...] = newv


def _emit_body(gs_ref, gt_ref, u_ref, v_ref, o_ref):
    m = _dotm(gs_ref, gt_ref)
    o_ref[...] = m * u_ref[...] * v_ref[...]


def _sinkhorn(gs, gt, rows=512):
    nt = N // rows
    byrow = lambda t: (t, 0)
    full = lambda t: (0, 0)
    v0 = pl.pallas_call(
        _colsum0_body,
        grid=(nt,),
        in_specs=[pl.BlockSpec((rows, D), byrow),
                  pl.BlockSpec((N, D), full)],
        out_specs=pl.BlockSpec((1, N), full),
        out_shape=jax.ShapeDtypeStruct((1, N), jnp.float32),
        scratch_shapes=[pltpu.VMEM((1, N), jnp.float32)],
    )(gs, gt)

    u, v = pl.pallas_call(
        functools.partial(_uv_body, rows=rows),
        grid=(2, nt),
        in_specs=[pl.BlockSpec((rows, D), lambda p, t: (t, 0)),
                  pl.BlockSpec((N, D), lambda p, t: (0, 0)),
                  pl.BlockSpec((1, N), lambda p, t: (0, 0))],
        out_specs=[pl.BlockSpec((rows, 1), lambda p, t: (t, 0)),
                   pl.BlockSpec((1, N), lambda p, t: (0, 0))],
        out_shape=[jax.ShapeDtypeStruct((N, 1), jnp.float32),
                   jax.ShapeDtypeStruct((1, N), jnp.float32)],
        scratch_shapes=[pltpu.VMEM((1, N), jnp.float32),
                        pltpu.VMEM((1, N), jnp.float32)],
    )(gs, gt, v0)

    return pl.pallas_call(
        _emit_body,
        grid=(nt,),
        in_specs=[pl.BlockSpec((rows, D), byrow),
                  pl.BlockSpec((N, D), full),
                  pl.BlockSpec((rows, 1), byrow),
                  pl.BlockSpec((1, N), full)],
        out_specs=pl.BlockSpec((rows, N), byrow),
        out_shape=jax.ShapeDtypeStruct((N, N), jnp.float32),
    )(gs, gt, u, v)


_SC_MESH = plsc.VectorSubcoreMesh(core_axis_name="c", subcore_axis_name="s")
_F32 = jnp.float32


def _agg_kernel_body(x_hbm, sl_hbm, dl_hbm, sg_hbm, dg_hbm, z_hbm,
                     af_hbm, ab_hbm,
                     sva, dva, sgva, dgva, svb, dvb, sgvb, dgvb,
                     r1a, r2a, r1b, r2b, accf, accb,
                     s1a, s2a, s1b, s2b):
    c = lax.axis_index("c")
    w = lax.axis_index("s")
    sl = pl.ds(w * ROWS_PT, ROWS_PT)

    pltpu.sync_copy(z_hbm, accf.at[sl])
    pltpu.sync_copy(z_hbm, accb.at[sl])
    plsc.subcore_barrier()

    ebase = c * E + w * EPT

    def load_idx(k, svx, dvx, sgvx, dgvx):
        base = ebase + k * CHA
        pltpu.sync_copy(sl_hbm.at[pl.ds(base, CHA)], svx)
        pltpu.sync_copy(dl_hbm.at[pl.ds(base, CHA)], dvx)
        pltpu.sync_copy(sg_hbm.at[pl.ds(base, CHA)], sgvx)
        pltpu.sync_copy(dg_hbm.at[pl.ds(base, CHA)], dgvx)

    def start(sgvx, dgvx, r1x, r2x, s1x, s2x):
        pltpu.async_copy(x_hbm.at[sgvx], r1x, s1x)
        pltpu.async_copy(x_hbm.at[dgvx], r2x, s2x)

    def finish(svx, dvx, sgvx, dgvx, r1x, r2x, s1x, s2x):
        pltpu.make_async_copy(x_hbm.at[sgvx], r1x, s1x).wait()
        pltpu.sync_copy(r1x, accf.at[dvx], add=True)
        pltpu.make_async_copy(x_hbm.at[dgvx], r2x, s2x).wait()
        pltpu.sync_copy(r2x, accb.at[svx], add=True)

    A = (sva, dva, sgva, dgva)
    B = (svb, dvb, sgvb, dgvb)
    RA = (r1a, r2a, s1a, s2a)
    RB = (r1b, r2b, s1b, s2b)

    load_idx(0, *A)
    start(sgva, dgva, *RA)

    @pl.loop(0, NCHUNKA - 2, step=2)
    def _(k):
        load_idx(k + 1, *B)
        start(sgvb, dgvb, *RB)
        finish(*A, *RA)
        load_idx(k + 2, *A)
        start(sgva, dgva, *RA)
        finish(*B, *RB)

    load_idx(NCHUNKA - 1, *B)
    start(sgvb, dgvb, *RB)
    finish(*A, *RA)
    finish(*B, *RB)

    plsc.subcore_barrier()

    orow = pl.ds(c * N + w * ROWS_PT, ROWS_PT)
    pltpu.sync_copy(accf.at[sl], af_hbm.at[orow])
    pltpu.sync_copy(accb.at[sl], ab_hbm.at[orow])


_agg = pl.kernel(
    _agg_kernel_body,
    out_type=[jax.ShapeDtypeStruct((NN, D), _F32),
              jax.ShapeDtypeStruct((NN, D), _F32)],
    mesh=_SC_MESH,
    scratch_types=(
        [pltpu.VMEM((CHA,), jnp.int32)] * 8 +
        [pltpu.VMEM((CHA, D), _F32)] * 4 +
        [pltpu.VMEM_SHARED((N, D), _F32),
         pltpu.VMEM_SHARED((N, D), _F32)] +
        [pltpu.SemaphoreType.DMA] * 4
    ),
)


def _counts_kernel_body(sl_hbm, dl_hbm, z_hbm, o_hbm, dep_hbm, cf_hbm, cb_hbm,
                        sv, dv, ones_v, cntf, cntb):
    del dep_hbm
    c = lax.axis_index("c")
    w = lax.axis_index("s")
    sl = pl.ds(w * ROWS_PT, ROWS_PT)

    pltpu.sync_copy(z_hbm, cntf.at[sl])
    pltpu.sync_copy(z_hbm, cntb.at[sl])
    pltpu.sync_copy(o_hbm, ones_v)
    plsc.subcore_barrier()

    ebase = c * E + w * EPT

    @pl.loop(0, NCHUNK)
    def _(k):
        base = ebase + k * CH
        pltpu.sync_copy(sl_hbm.at[pl.ds(base, CH)], sv)
        pltpu.sync_copy(dl_hbm.at[pl.ds(base, CH)], dv)
        pltpu.sync_copy(ones_v, cntf.at[dv], add=True)
        pltpu.sync_copy(ones_v, cntb.at[sv], add=True)

    plsc.subcore_barrier()

    orow = pl.ds(c * N + w * ROWS_PT, ROWS_PT)
    pltpu.sync_copy(cntf.at[sl], cf_hbm.at[orow])
    pltpu.sync_copy(cntb.at[sl], cb_hbm.at[orow])


_counts = pl.kernel(
    _counts_kernel_body,
    out_type=[jax.ShapeDtypeStruct((NN, D), _F32),
              jax.ShapeDtypeStruct((NN, D), _F32)],
    mesh=_SC_MESH,
    scratch_types=[
        pltpu.VMEM((CH,), jnp.int32),
        pltpu.VMEM((CH,), jnp.int32),
        pltpu.VMEM((CH, D), _F32),
        pltpu.VMEM_SHARED((N, D), _F32),
        pltpu.VMEM_SHARED((N, D), _F32),
    ],
)


def kernel(x_s, x_t, edges, edget, Hs, Gs, Ht, Gt, dedges, dws, dedget, dwt,
           l1W0, l2W0, rW0, rb0, l1W1, l2W1, rW1, rb1, finW, finb):
    x = jnp.concatenate([x_s, x_t], axis=0)
    src_l = jnp.concatenate([edges[0], edget[0]]).astype(jnp.int32)
    dst_l = jnp.concatenate([edges[1], edget[1]]).astype(jnp.int32)
    src_g = jnp.concatenate([edges[0], edget[0] + N]).astype(jnp.int32)
    dst_g = jnp.concatenate([edges[1], edget[1] + N]).astype(jnp.int32)
    z = jnp.zeros((ROWS_PT, D), jnp.float32)
    on = jnp.ones((CH, D), jnp.float32)

    af, ab = _agg(x, src_l, dst_l, src_g, dst_g, z)
    cf, cb = _counts(src_l, dst_l, z, on, af)
    h1 = _layer(x, af, ab, cf, cb, l1W0, l2W0, rW0, rb0[None, :])

    af, ab = _agg(h1, src_l, dst_l, src_g, dst_g, z)
    h2 = _layer(h1, af, ab, cf, cb, l1W1, l2W1, rW1, rb1[None, :])

    f0, f1, f2 = finW[:D], finW[D:2 * D], finW[2 * D:]
    h = _final(x, h1, h2, f0, f1, f2, finb[None, :])

    return _sinkhorn(h[:N], h[N:])


# speedup vs baseline: 5.1677x; 1.0319x over previous
import functools

import jax
from jax import lax
import jax.numpy as jnp
from jax.experimental import pallas as pl
from jax.experimental.pallas import tpu as pltpu
from jax.experimental.pallas import tpu_sc as plsc

N = 4096
NN = 2 * N
D = 128
E = 65536
EPS2 = 2e-10

NTILES = 16
EPT = E // NTILES
CH = 128
NCHUNK = EPT // CH
CHA = 64
NCHUNKA = EPT // CHA
ROWS_PT = N // NTILES


def _layer_body(x_ref, af_ref, ab_ref, c1_ref, c2_ref, w1_ref, w2_ref, wr_ref,
                br_ref, o_ref):
    x = x_ref[...]
    c1 = jnp.maximum(c1_ref[...][:, :1], 1.0)
    c2 = jnp.maximum(c2_ref[...][:, :1], 1.0)
    af = af_ref[...] / c1
    ab = ab_ref[...] / c2
    acc = jax.lax.dot_general(x, wr_ref[...], (((1,), (0,)), ((), ())),
                              preferred_element_type=jnp.float32)
    acc += jax.lax.dot_general(af, w1_ref[...], (((1,), (0,)), ((), ())),
                               preferred_element_type=jnp.float32)
    acc += jax.lax.dot_general(ab, w2_ref[...], (((1,), (0,)), ((), ())),
                               preferred_element_type=jnp.float32)
    o_ref[...] = jnp.maximum(acc + br_ref[...], 0.0)


def _layer(x, af, ab, c1, c2, w1, w2, wr, br, rows=512):
    t = NN // rows
    full = lambda i: (0, 0)
    byrow = lambda i: (i, 0)
    return pl.pallas_call(
        _layer_body,
        grid=(t,),
        in_specs=[
            pl.BlockSpec((rows, D), byrow),
            pl.BlockSpec((rows, D), byrow),
            pl.BlockSpec((rows, D), byrow),
            pl.BlockSpec((rows, D), byrow),
            pl.BlockSpec((rows, D), byrow),
            pl.BlockSpec((D, D), full),
            pl.BlockSpec((D, D), full),
            pl.BlockSpec((D, D), full),
            pl.BlockSpec((1, D), full),
        ],
        out_specs=pl.BlockSpec((rows, D), byrow),
        out_shape=jax.ShapeDtypeStruct((NN, D), jnp.float32),
    )(x, af, ab, c1, c2, w1, w2, wr, br)


def _final_body(x_ref, h1_ref, h2_ref, f0_ref, f1_ref, f2_ref, fb_ref, o_ref):
    acc = jax.lax.dot_general(x_ref[...], f0_ref[...], (((1,), (0,)), ((), ())),
                              preferred_element_type=jnp.float32)
    acc += jax.lax.dot_general(h1_ref[...], f1_ref[...], (((1,), (0,)), ((), ())),
                               preferred_element_type=jnp.float32)
    acc += jax.lax.dot_general(h2_ref[...], f2_ref[...], (((1,), (0,)), ((), ())),
                               preferred_element_type=jnp.float32)
    o_ref[...] = ((acc + fb_ref[...]) * 1.6986724).astype(jnp.bfloat16)


def _final(x, h1, h2, f0, f1, f2, fb, rows=512):
    t = NN // rows
    full = lambda i: (0, 0)
    byrow = lambda i: (i, 0)
    return pl.pallas_call(
        _final_body,
        grid=(t,),
        in_specs=[
            pl.BlockSpec((rows, D), byrow),
            pl.BlockSpec((rows, D), byrow),
            pl.BlockSpec((rows, D), byrow),
            pl.BlockSpec((D, D), full),
            pl.BlockSpec((D, D), full),
            pl.BlockSpec((D, D), full),
            pl.BlockSpec((1, D), full),
        ],
        out_specs=pl.BlockSpec((rows, D), byrow),
        out_shape=jax.ShapeDtypeStruct((NN, D), jnp.bfloat16),
    )(x, h1, h2, f0, f1, f2, fb)


def _dotm(a_ref, b_ref):
    return jnp.exp2(jax.lax.dot_general(a_ref[...], b_ref[...],
                                        (((1,), (1,)), ((), ())),
                                        preferred_element_type=jnp.float32))


def _colsum0_body(gs_ref, gt_ref, v0_ref, acc_ref):
    t = pl.program_id(0)
    nt = pl.num_programs(0)
    m = _dotm(gs_ref, gt_ref)

    @pl.when(t == 0)
    def _():
        acc_ref[...] = jnp.zeros_like(acc_ref)

    acc_ref[...] += jnp.sum(m, axis=0, keepdims=True)

    @pl.when(t == nt - 1)
    def _():
        v0_ref[...] = 1.0 / acc_ref[...]


def _uv_body(gs_ref, gt_ref, v0_ref, u_ref, v_ref, vv_ref, acc_ref, *, rows):
    p = pl.program_id(0)
    t = pl.program_id(1)
    nt = pl.num_programs(1)

    @pl.when((p == 0) & (t == 0))
    def _():
        vv_ref[...] = v0_ref[...]

    @pl.when(t == 0)
    def _():
        acc_ref[...] = jnp.zeros_like(acc_ref)

    m = _dotm(gs_ref, gt_ref)
    u_t = 1.0 / jnp.sum(m * vv_ref[...], axis=1, keepdims=True)
    acc_ref[...] += jnp.sum(m * u_t, axis=0, keepdims=True)
    u_ref[...] = u_t

    @pl.when(t == nt - 1)
    def _():
        newv = 1.0 / acc_ref[...]
        vv_ref[...] = newv
        v_ref[...] = newv


def _emit_body(gs_ref, gt_ref, u_ref, v_ref, o_ref):
    m = _dotm(gs_ref, gt_ref)
    o_ref[...] = m * u_ref[...] * v_ref[...]


def _sinkhorn(gs, gt, rows=512):
    nt = N // rows
    byrow = lambda t: (t, 0)
    full = lambda t: (0, 0)
    v0 = pl.pallas_call(
        _colsum0_body,
        grid=(nt,),
        in_specs=[pl.BlockSpec((rows, D), byrow),
                  pl.BlockSpec((N, D), full)],
        out_specs=pl.BlockSpec((1, N), full),
        out_shape=jax.ShapeDtypeStruct((1, N), jnp.float32),
        scratch_shapes=[pltpu.VMEM((1, N), jnp.float32)],
    )(gs, gt)

    u, v = pl.pallas_call(
        functools.partial(_uv_body, rows=rows),
        grid=(2, nt),
        in_specs=[pl.BlockSpec((rows, D), lambda p, t: (t, 0)),
                  pl.BlockSpec((N, D), lambda p, t: (0, 0)),
                  pl.BlockSpec((1, N), lambda p, t: (0, 0))],
        out_specs=[pl.BlockSpec((rows, 1), lambda p, t: (t, 0)),
                   pl.BlockSpec((1, N), lambda p, t: (0, 0))],
        out_shape=[jax.ShapeDtypeStruct((N, 1), jnp.float32),
                   jax.ShapeDtypeStruct((1, N), jnp.float32)],
        scratch_shapes=[pltpu.VMEM((1, N), jnp.float32),
                        pltpu.VMEM((1, N), jnp.float32)],
    )(gs, gt, v0)

    return pl.pallas_call(
        _emit_body,
        grid=(nt,),
        in_specs=[pl.BlockSpec((rows, D), byrow),
                  pl.BlockSpec((N, D), full),
                  pl.BlockSpec((rows, 1), byrow),
                  pl.BlockSpec((1, N), full)],
        out_specs=pl.BlockSpec((rows, N), byrow),
        out_shape=jax.ShapeDtypeStruct((N, N), jnp.float32),
    )(gs, gt, u, v)


_SC_MESH = plsc.VectorSubcoreMesh(core_axis_name="c", subcore_axis_name="s")
_F32 = jnp.float32


_NSETS = 3
_NLOOP = (NCHUNKA - 1) // _NSETS - 1


def _agg_kernel_body(x_hbm, sl_hbm, dl_hbm, sg_hbm, dg_hbm, z_hbm,
                     af_hbm, ab_hbm, *rest):
    sets = []
    for i in range(_NSETS):
        sets.append(rest[i * 10:(i + 1) * 10])
    accf, accb = rest[10 * _NSETS], rest[10 * _NSETS + 1]

    c = lax.axis_index("c")
    w = lax.axis_index("s")
    sl = pl.ds(w * ROWS_PT, ROWS_PT)

    pltpu.sync_copy(z_hbm, accf.at[sl])
    pltpu.sync_copy(z_hbm, accb.at[sl])
    plsc.subcore_barrier()

    ebase = c * E + w * EPT

    def li_g(k, S):
        sv, dv, sgv, dgv, r1, r2, s1, s2, t1, t2 = S
        base = ebase + k * CHA
        pltpu.sync_copy(sl_hbm.at[pl.ds(base, CHA)], sv)
        pltpu.sync_copy(dl_hbm.at[pl.ds(base, CHA)], dv)
        pltpu.sync_copy(sg_hbm.at[pl.ds(base, CHA)], sgv)
        pltpu.sync_copy(dg_hbm.at[pl.ds(base, CHA)], dgv)
        pltpu.async_copy(x_hbm.at[sgv], r1, s1)
        pltpu.async_copy(x_hbm.at[dgv], r2, s2)

    def wg_sc(S):
        sv, dv, sgv, dgv, r1, r2, s1, s2, t1, t2 = S
        pltpu.make_async_copy(x_hbm.at[sgv], r1, s1).wait()
        pltpu.make_async_copy(x_hbm.at[dgv], r2, s2).wait()
        pltpu.async_copy(r1, accf.at[dv], t1, add=True)
        pltpu.async_copy(r2, accb.at[sv], t2, add=True)

    def ws(S):
        sv, dv, sgv, dgv, r1, r2, s1, s2, t1, t2 = S
        pltpu.make_async_copy(x_hbm.at[pl.ds(0, CHA)], r1, t1).wait()
        pltpu.make_async_copy(x_hbm.at[pl.ds(0, CHA)], r2, t2).wait()

    for i in range(_NSETS):
        li_g(i, sets[i])

    @pl.loop(0, _NLOOP)
    def _(it):
        j = _NSETS * it
        for i in range(_NSETS):
            wg_sc(sets[i])
        for i in range(_NSETS):
            ws(sets[i])
            li_g(j + _NSETS + i, sets[i])

    done = _NSETS + _NLOOP * _NSETS
    for i in range(_NSETS):
        wg_sc(sets[i])
    for k in range(done, NCHUNKA):
        i = (k - done) % _NSETS
        ws(sets[i])
        li_g(k, sets[i])
        wg_sc(sets[i])
    for i in range(_NSETS):
        ws(sets[i])

    plsc.subcore_barrier()

    orow = pl.ds(c * N + w * ROWS_PT, ROWS_PT)
    pltpu.sync_copy(accf.at[sl], af_hbm.at[orow])
    pltpu.sync_copy(accb.at[sl], ab_hbm.at[orow])


_agg = pl.kernel(
    _agg_kernel_body,
    out_type=[jax.ShapeDtypeStruct((NN, D), _F32),
              jax.ShapeDtypeStruct((NN, D), _F32)],
    mesh=_SC_MESH,
    scratch_types=(
        ([pltpu.VMEM((CHA,), jnp.int32)] * 4 +
         [pltpu.VMEM((CHA, D), _F32)] * 2 +
         [pltpu.SemaphoreType.DMA] * 4
         ) * _NSETS +
        [pltpu.VMEM_SHARED((N, D), _F32),
         pltpu.VMEM_SHARED((N, D), _F32)]
    ),
)


def _counts_kernel_body(sl_hbm, dl_hbm, z_hbm, o_hbm, dep_hbm, cf_hbm, cb_hbm,
                        sv, dv, ones_v, cntf, cntb):
    del dep_hbm
    c = lax.axis_index("c")
    w = lax.axis_index("s")
    sl = pl.ds(w * ROWS_PT, ROWS_PT)

    pltpu.sync_copy(z_hbm, cntf.at[sl])
    pltpu.sync_copy(z_hbm, cntb.at[sl])
    pltpu.sync_copy(o_hbm, ones_v)
    plsc.subcore_barrier()

    ebase = c * E + w * EPT

    @pl.loop(0, NCHUNK)
    def _(k):
        base = ebase + k * CH
        pltpu.sync_copy(sl_hbm.at[pl.ds(base, CH)], sv)
        pltpu.sync_copy(dl_hbm.at[pl.ds(base, CH)], dv)
        pltpu.sync_copy(ones_v, cntf.at[dv], add=True)
        pltpu.sync_copy(ones_v, cntb.at[sv], add=True)

    plsc.subcore_barrier()

    orow = pl.ds(c * N + w * ROWS_PT, ROWS_PT)
    pltpu.sync_copy(cntf.at[sl], cf_hbm.at[orow])
    pltpu.sync_copy(cntb.at[sl], cb_hbm.at[orow])


_counts = pl.kernel(
    _counts_kernel_body,
    out_type=[jax.ShapeDtypeStruct((NN, D), _F32),
              jax.ShapeDtypeStruct((NN, D), _F32)],
    mesh=_SC_MESH,
    scratch_types=[
        pltpu.VMEM((CH,), jnp.int32),
        pltpu.VMEM((CH,), jnp.int32),
        pltpu.VMEM((CH, D), _F32),
        pltpu.VMEM_SHARED((N, D), _F32),
        pltpu.VMEM_SHARED((N, D), _F32),
    ],
)


def kernel(x_s, x_t, edges, edget, Hs, Gs, Ht, Gt, dedges, dws, dedget, dwt,
           l1W0, l2W0, rW0, rb0, l1W1, l2W1, rW1, rb1, finW, finb):
    x = jnp.concatenate([x_s, x_t], axis=0)
    src_l = jnp.concatenate([edges[0], edget[0]]).astype(jnp.int32)
    dst_l = jnp.concatenate([edges[1], edget[1]]).astype(jnp.int32)
    src_g = jnp.concatenate([edges[0], edget[0] + N]).astype(jnp.int32)
    dst_g = jnp.concatenate([edges[1], edget[1] + N]).astype(jnp.int32)
    z = jnp.zeros((ROWS_PT, D), jnp.float32)
    on = jnp.ones((CH, D), jnp.float32)

    af, ab = _agg(x, src_l, dst_l, src_g, dst_g, z)
    cf, cb = _counts(src_l, dst_l, z, on, af)
    h1 = _layer(x, af, ab, cf, cb, l1W0, l2W0, rW0, rb0[None, :])

    af, ab = _agg(h1, src_l, dst_l, src_g, dst_g, z)
    h2 = _layer(h1, af, ab, cf, cb, l1W1, l2W1, rW1, rb1[None, :])

    f0, f1, f2 = finW[:D], finW[D:2 * D], finW[2 * D:]
    h = _final(x, h1, h2, f0, f1, f2, finb[None, :])

    return _sinkhorn(h[:N], h[N:])


# speedup vs baseline: 5.6271x; 1.0889x over previous
import dataclasses
import functools

import jax
from jax import lax
import jax.numpy as jnp
from jax.experimental import pallas as pl
from jax.experimental.pallas import tpu as pltpu
from jax.experimental.pallas import tpu_sc as plsc

N = 4096
NN = 2 * N
D = 128
E = 65536
EPS2 = 2e-10

NTILES = 16
EPT = E // NTILES
CH = 128
NCHUNK = EPT // CH
CHA = 64
NCHUNKA = EPT // CHA
ROWS_PT = N // NTILES


def _layer_body(x_ref, af_ref, ab_ref, c1_ref, c2_ref, w1_ref, w2_ref, wr_ref,
                br_ref, o_ref):
    x = x_ref[...]
    rows = x.shape[0]
    i = pl.program_id(0)
    pk = rows // 128

    def bychunk(cref, aref):
        parts = []
        for q in range(pk):
            ct = cref[pl.ds(i * pk + q, 1), :]
            col = jnp.maximum(jnp.transpose(ct), 1.0)
            parts.append(aref[pl.ds(q * 128, 128), :] / col)
        return jnp.concatenate(parts, axis=0)

    af = bychunk(c1_ref, af_ref)
    ab = bychunk(c2_ref, ab_ref)
    acc = jax.lax.dot_general(x, wr_ref[...], (((1,), (0,)), ((), ())),
                              preferred_element_type=jnp.float32)
    acc += jax.lax.dot_general(af, w1_ref[...], (((1,), (0,)), ((), ())),
                               preferred_element_type=jnp.float32)
    acc += jax.lax.dot_general(ab, w2_ref[...], (((1,), (0,)), ((), ())),
                               preferred_element_type=jnp.float32)
    o_ref[...] = jnp.maximum(acc + br_ref[...], 0.0)


def _layer(x, af, ab, c1, c2, w1, w2, wr, br, rows=512):
    t = NN // rows
    full = lambda i: (0, 0)
    byrow = lambda i: (i, 0)
    return pl.pallas_call(
        _layer_body,
        grid=(t,),
        in_specs=[
            pl.BlockSpec((rows, D), byrow),
            pl.BlockSpec((rows, D), byrow),
            pl.BlockSpec((rows, D), byrow),
            pl.BlockSpec((NN // 128, 128), full),
            pl.BlockSpec((NN // 128, 128), full),
            pl.BlockSpec((D, D), full),
            pl.BlockSpec((D, D), full),
            pl.BlockSpec((D, D), full),
            pl.BlockSpec((1, D), full),
        ],
        out_specs=pl.BlockSpec((rows, D), byrow),
        out_shape=jax.ShapeDtypeStruct((NN, D), jnp.float32),
    )(x, af, ab, c1, c2, w1, w2, wr, br)


def _final_body(x_ref, h1_ref, h2_ref, f0_ref, f1_ref, f2_ref, fb_ref, o_ref):
    acc = jax.lax.dot_general(x_ref[...], f0_ref[...], (((1,), (0,)), ((), ())),
                              preferred_element_type=jnp.float32)
    acc += jax.lax.dot_general(h1_ref[...], f1_ref[...], (((1,), (0,)), ((), ())),
                               preferred_element_type=jnp.float32)
    acc += jax.lax.dot_general(h2_ref[...], f2_ref[...], (((1,), (0,)), ((), ())),
                               preferred_element_type=jnp.float32)
    o_ref[...] = ((acc + fb_ref[...]) * 1.6986724).astype(jnp.bfloat16)


def _final(x, h1, h2, f0, f1, f2, fb, rows=512):
    t = NN // rows
    full = lambda i: (0, 0)
    byrow = lambda i: (i, 0)
    return pl.pallas_call(
        _final_body,
        grid=(t,),
        in_specs=[
            pl.BlockSpec((rows, D), byrow),
            pl.BlockSpec((rows, D), byrow),
            pl.BlockSpec((rows, D), byrow),
            pl.BlockSpec((D, D), full),
            pl.BlockSpec((D, D), full),
            pl.BlockSpec((D, D), full),
            pl.BlockSpec((1, D), full),
        ],
        out_specs=pl.BlockSpec((rows, D), byrow),
        out_shape=jax.ShapeDtypeStruct((NN, D), jnp.bfloat16),
    )(x, h1, h2, f0, f1, f2, fb)


def _dotm(a_ref, b_ref):
    return jnp.exp2(jax.lax.dot_general(a_ref[...], b_ref[...],
                                        (((1,), (1,)), ((), ())),
                                        preferred_element_type=jnp.float32))


def _colsum0_body(gs_ref, gt_ref, v0_ref, acc_ref):
    t = pl.program_id(0)
    nt = pl.num_programs(0)
    m = _dotm(gs_ref, gt_ref)

    @pl.when(t == 0)
    def _():
        acc_ref[...] = jnp.zeros_like(acc_ref)

    acc_ref[...] += jnp.sum(m, axis=0, keepdims=True)

    @pl.when(t == nt - 1)
    def _():
        v0_ref[...] = 1.0 / acc_ref[...]


def _uv_body(gs_ref, gt_ref, v0_ref, u_ref, v_ref, vv_ref, acc_ref, *, rows):
    p = pl.program_id(0)
    t = pl.program_id(1)
    nt = pl.num_programs(1)

    @pl.when((p == 0) & (t == 0))
    def _():
        vv_ref[...] = v0_ref[...]

    @pl.when(t == 0)
    def _():
        acc_ref[...] = jnp.zeros_like(acc_ref)

    m = _dotm(gs_ref, gt_ref)
    u_t = 1.0 / jnp.sum(m * vv_ref[...], axis=1, keepdims=True)
    acc_ref[...] += jnp.sum(m * u_t, axis=0, keepdims=True)
    u_ref[...] = u_t

    @pl.when(t == nt - 1)
    def _():
        newv = 1.0 / acc_ref[...]
        vv_ref[...] = newv
        v_ref[...] = newv


def _emit_body(gs_ref, gt_ref, u_ref, v_ref, o_ref):
    m = _dotm(gs_ref, gt_ref)
    o_ref[...] = m * u_ref[...] * v_ref[...]


def _sinkhorn(gs, gt, rows=512):
    nt = N // rows
    byrow = lambda t: (t, 0)
    full = lambda t: (0, 0)
    v0 = pl.pallas_call(
        _colsum0_body,
        grid=(nt,),
        in_specs=[pl.BlockSpec((rows, D), byrow),
                  pl.BlockSpec((N, D), full)],
        out_specs=pl.BlockSpec((1, N), full),
        out_shape=jax.ShapeDtypeStruct((1, N), jnp.float32),
        scratch_shapes=[pltpu.VMEM((1, N), jnp.float32)],
    )(gs, gt)

    u, v = pl.pallas_call(
        functools.partial(_uv_body, rows=rows),
        grid=(2, nt),
        in_specs=[pl.BlockSpec((rows, D), lambda p, t: (t, 0)),
                  pl.BlockSpec((N, D), lambda p, t: (0, 0)),
                  pl.BlockSpec((1, N), lambda p, t: (0, 0))],
        out_specs=[pl.BlockSpec((rows, 1), lambda p, t: (t, 0)),
                   pl.BlockSpec((1, N), lambda p, t: (0, 0))],
        out_shape=[jax.ShapeDtypeStruct((N, 1), jnp.float32),
                   jax.ShapeDtypeStruct((1, N), jnp.float32)],
        scratch_shapes=[pltpu.VMEM((1, N), jnp.float32),
                        pltpu.VMEM((1, N), jnp.float32)],
    )(gs, gt, v0)

    return pl.pallas_call(
        _emit_body,
        grid=(nt,),
        in_specs=[pl.BlockSpec((rows, D), byrow),
                  pl.BlockSpec((N, D), full),
                  pl.BlockSpec((rows, 1), byrow),
                  pl.BlockSpec((1, N), full)],
        out_specs=pl.BlockSpec((rows, N), byrow),
        out_shape=jax.ShapeDtypeStruct((N, N), jnp.float32),
    )(gs, gt, u, v)


_SC_MESH = plsc.VectorSubcoreMesh(core_axis_name="c", subcore_axis_name="s")
_F32 = jnp.float32


_NSETS = 3
_NLOOP = (NCHUNKA - 1) // _NSETS - 1


def _agg_kernel_body(x_hbm, sl_hbm, dl_hbm, sg_hbm, dg_hbm, z_hbm,
                     af_hbm, ab_hbm, *rest):
    sets = []
    for i in range(_NSETS):
        sets.append(rest[i * 10:(i + 1) * 10])
    accf, accb = rest[10 * _NSETS], rest[10 * _NSETS + 1]

    c = lax.axis_index("c")
    w = lax.axis_index("s")
    sl = pl.ds(w * ROWS_PT, ROWS_PT)

    pltpu.sync_copy(z_hbm, accf.at[sl])
    pltpu.sync_copy(z_hbm, accb.at[sl])
    plsc.subcore_barrier()

    ebase = c * E + w * EPT

    def li_g(k, S):
        sv, dv, sgv, dgv, r1, r2, s1, s2, t1, t2 = S
        base = ebase + k * CHA
        pltpu.sync_copy(sl_hbm.at[pl.ds(base, CHA)], sv)
        pltpu.sync_copy(dl_hbm.at[pl.ds(base, CHA)], dv)
        pltpu.sync_copy(sg_hbm.at[pl.ds(base, CHA)], sgv)
        pltpu.sync_copy(dg_hbm.at[pl.ds(base, CHA)], dgv)
        pltpu.async_copy(x_hbm.at[sgv], r1, s1)
        pltpu.async_copy(x_hbm.at[dgv], r2, s2)

    def wg_sc(S):
        sv, dv, sgv, dgv, r1, r2, s1, s2, t1, t2 = S
        pltpu.make_async_copy(x_hbm.at[sgv], r1, s1).wait()
        pltpu.make_async_copy(x_hbm.at[dgv], r2, s2).wait()
        pltpu.async_copy(r1, accf.at[dv], t1, add=True)
        pltpu.async_copy(r2, accb.at[sv], t2, add=True)

    def ws(S):
        sv, dv, sgv, dgv, r1, r2, s1, s2, t1, t2 = S
        pltpu.make_async_copy(x_hbm.at[pl.ds(0, CHA)], r1, t1).wait()
        pltpu.make_async_copy(x_hbm.at[pl.ds(0, CHA)], r2, t2).wait()

    for i in range(_NSETS):
        li_g(i, sets[i])

    @pl.loop(0, _NLOOP)
    def _(it):
        j = _NSETS * it
        for i in range(_NSETS):
            wg_sc(sets[i])
        for i in range(_NSETS):
            ws(sets[i])
            li_g(j + _NSETS + i, sets[i])

    done = _NSETS + _NLOOP * _NSETS
    for i in range(_NSETS):
        wg_sc(sets[i])
    for k in range(done, NCHUNKA):
        i = (k - done) % _NSETS
        ws(sets[i])
        li_g(k, sets[i])
        wg_sc(sets[i])
    for i in range(_NSETS):
        ws(sets[i])

    plsc.subcore_barrier()

    orow = pl.ds(c * N + w * ROWS_PT, ROWS_PT)
    pltpu.sync_copy(accf.at[sl], af_hbm.at[orow])
    pltpu.sync_copy(accb.at[sl], ab_hbm.at[orow])


_agg = pl.kernel(
    _agg_kernel_body,
    out_type=[jax.ShapeDtypeStruct((NN, D), _F32),
              jax.ShapeDtypeStruct((NN, D), _F32)],
    mesh=_SC_MESH,
    scratch_types=(
        ([pltpu.VMEM((CHA,), jnp.int32)] * 4 +
         [pltpu.VMEM((CHA, D), _F32)] * 2 +
         [pltpu.SemaphoreType.DMA] * 4
         ) * _NSETS +
        [pltpu.VMEM_SHARED((N, D), _F32),
         pltpu.VMEM_SHARED((N, D), _F32)]
    ),
)


def _counts_kernel_body(sl_hbm, dl_hbm, z_hbm, o_hbm, dep_hbm, cf_hbm, cb_hbm,
                        sv, dv, ones_v, cntf, cntb):
    del dep_hbm
    c = lax.axis_index("c")
    w = lax.axis_index("s")
    sl = pl.ds(w * ROWS_PT, ROWS_PT)

    pltpu.sync_copy(z_hbm, cntf.at[sl])
    pltpu.sync_copy(z_hbm, cntb.at[sl])
    pltpu.sync_copy(o_hbm, ones_v)
    plsc.subcore_barrier()

    ebase = c * E + w * EPT

    @pl.loop(0, NCHUNK)
    def _(k):
        base = ebase + k * CH
        pltpu.sync_copy(sl_hbm.at[pl.ds(base, CH)], sv)
        pltpu.sync_copy(dl_hbm.at[pl.ds(base, CH)], dv)
        pltpu.sync_copy(ones_v, cntf.at[dv], add=True)
        pltpu.sync_copy(ones_v, cntb.at[sv], add=True)

    plsc.subcore_barrier()

    orow = pl.ds(c * N + w * ROWS_PT, ROWS_PT)
    pltpu.sync_copy(cntf.at[sl], cf_hbm.at[orow])
    pltpu.sync_copy(cntb.at[sl], cb_hbm.at[orow])


def _counts_hist_body(sl_hbm, dl_hbm, dep_hbm, cf_hbm, cb_hbm,
                      sv, dv, hist_f, hist_b, rbuf, pk_f, pk_b,
                      stage_f, stage_b):
    del dep_hbm
    c = lax.axis_index("c")
    w = lax.axis_index("s")
    zero16 = jnp.zeros((16,), jnp.float32)
    one16 = jnp.full((16,), 1.0, jnp.float32)

    for i in range(N // 16):
        hist_f[pl.ds(16 * i, 16)] = zero16
        hist_b[pl.ds(16 * i, 16)] = zero16

    ebase = c * E + w * EPT

    @pl.loop(0, NCHUNK)
    def _(k):
        base = ebase + k * CH
        pltpu.sync_copy(sl_hbm.at[pl.ds(base, CH)], sv)
        pltpu.sync_copy(dl_hbm.at[pl.ds(base, CH)], dv)
        for i in range(CH // 16):
            plsc.addupdate_scatter(hist_f, [dv[pl.ds(16 * i, 16)]], one16)
            plsc.addupdate_scatter(hist_b, [sv[pl.ds(16 * i, 16)]], one16)

    for r in range(NTILES):
        seg = pl.ds(r * ROWS_PT, ROWS_PT)
        dstseg = pl.ds(w * ROWS_PT, ROWS_PT)
        pltpu.sync_copy(hist_f.at[seg], stage_f.at[r, dstseg])
        pltpu.sync_copy(hist_b.at[seg], stage_b.at[r, dstseg])
    plsc.subcore_barrier()

    for (stage, pk) in ((stage_f, pk_f), (stage_b, pk_b)):
        pltpu.sync_copy(stage.at[w], rbuf)
        for g in range(ROWS_PT // 16):
            v = rbuf[pl.ds(g * 16, 16)]
            for t in range(1, NTILES):
                v = v + rbuf[pl.ds(t * ROWS_PT + g * 16, 16)]
            pk[(g * 16) // 128, pl.ds((g * 16) % 128, 16)] = v

    prow = pl.ds(c * (N // 128) + w * (ROWS_PT // 128), ROWS_PT // 128)
    pltpu.sync_copy(pk_f, cf_hbm.at[prow])
    pltpu.sync_copy(pk_b, cb_hbm.at[prow])


_SC_CP = pltpu.CompilerParams()
if "needs_layout_passes" in pltpu.CompilerParams.__dataclass_fields__:
    _SC_CP = dataclasses.replace(_SC_CP, needs_layout_passes=False)

_counts_hist = pl.kernel(
    _counts_hist_body,
    out_type=[jax.ShapeDtypeStruct((NN // 128, 128), _F32),
              jax.ShapeDtypeStruct((NN // 128, 128), _F32)],
    mesh=_SC_MESH,
    compiler_params=_SC_CP,
    scratch_types=[
        pltpu.VMEM((CH,), jnp.int32),
        pltpu.VMEM((CH,), jnp.int32),
        pltpu.VMEM((N,), _F32),
        pltpu.VMEM((N,), _F32),
        pltpu.VMEM((N,), _F32),
        pltpu.VMEM((ROWS_PT // 128, 128), _F32),
        pltpu.VMEM((ROWS_PT // 128, 128), _F32),
        pltpu.VMEM_SHARED((NTILES, N), _F32),
        pltpu.VMEM_SHARED((NTILES, N), _F32),
    ],
)


_counts = pl.kernel(
    _counts_kernel_body,
    out_type=[jax.ShapeDtypeStruct((NN, D), _F32),
              jax.ShapeDtypeStruct((NN, D), _F32)],
    mesh=_SC_MESH,
    scratch_types=[
        pltpu.VMEM((CH,), jnp.int32),
        pltpu.VMEM((CH,), jnp.int32),
        pltpu.VMEM((CH, D), _F32),
        pltpu.VMEM_SHARED((N, D), _F32),
        pltpu.VMEM_SHARED((N, D), _F32),
    ],
)


def kernel(x_s, x_t, edges, edget, Hs, Gs, Ht, Gt, dedges, dws, dedget, dwt,
           l1W0, l2W0, rW0, rb0, l1W1, l2W1, rW1, rb1, finW, finb):
    x = jnp.concatenate([x_s, x_t], axis=0)
    src_l = jnp.concatenate([edges[0], edget[0]]).astype(jnp.int32)
    dst_l = jnp.concatenate([edges[1], edget[1]]).astype(jnp.int32)
    src_g = jnp.concatenate([edges[0], edget[0] + N]).astype(jnp.int32)
    dst_g = jnp.concatenate([edges[1], edget[1] + N]).astype(jnp.int32)
    z = jnp.zeros((ROWS_PT, D), jnp.float32)

    af, ab = _agg(x, src_l, dst_l, src_g, dst_g, z)
    cf, cb = _counts_hist(src_l, dst_l, af)
    h1 = _layer(x, af, ab, cf, cb, l1W0, l2W0, rW0, rb0[None, :])

    af, ab = _agg(h1, src_l, dst_l, src_g, dst_g, z)
    h2 = _layer(h1, af, ab, cf, cb, l1W1, l2W1, rW1, rb1[None, :])

    f0, f1, f2 = finW[:D], finW[D:2 * D], finW[2 * D:]
    h = _final(x, h1, h2, f0, f1, f2, finb[None, :])

    return _sinkhorn(h[:N], h[N:])
